# baseline (speedup 1.0000x reference)
import functools

import jax
import jax.numpy as jnp
from jax import lax
from jax.experimental import pallas as pl
from jax.experimental.pallas import tpu as pltpu
from jax.experimental.pallas import tpu_sc as plsc

B = 1
S = 2048
D = 1024
H = 8
HD = D // H
T = S * H
E = 8
K = 2
F = 512

TH = T // 2
SH = S // 2
BM = 1024
NBH = TH * K // BM + E
PS = NBH * BM
NW = 32
TPW = TH // NW
CH = 128
NCH = TPW // CH
RB = 8192

_GELU_C = 0.7978845608028654
_GC1 = -2.0 * _GELU_C * 1.4426950408889634
_GC2 = _GC1 * 0.044715


def _gelu(x):
    t = jnp.exp2((_GC1 + _GC2 * (x * x)) * x)
    return x / (1.0 + t)


def _mh_kernel(x_ref, w_ref, b_ref, o_ref):
    y = (
        jnp.dot(x_ref[...], w_ref[...], preferred_element_type=jnp.float32)
        + b_ref[...]
    )
    o_ref[...] = y.reshape(o_ref.shape)


def _mh_proj(x, w, b, bm=512):
    return pl.pallas_call(
        _mh_kernel,
        grid=(S // bm,),
        in_specs=[
            pl.BlockSpec((bm, D), lambda i: (i, 0)),
            pl.BlockSpec((D, D), lambda i: (0, 0)),
            pl.BlockSpec((1, D), lambda i: (0, 0)),
        ],
        out_specs=pl.BlockSpec((bm * H, HD), lambda i: (i, 0)),
        out_shape=jax.ShapeDtypeStruct((T, HD), jnp.float32),
    )(x, w, b.reshape(1, D))


def _router_kernel(y_ref, wg_ref, e0_ref, e1_ref, g0_ref, g1_ref):
    y = y_ref[...]
    f32 = jnp.float32
    logits = jnp.dot(y, wg_ref[...], preferred_element_type=f32)
    U8 = (jax.lax.broadcasted_iota(jnp.int32, (E, E), 0)
          < jax.lax.broadcasted_iota(jnp.int32, (E, E), 1)).astype(f32)
    icol = jax.lax.broadcasted_iota(jnp.int32, (E, 1), 0).astype(f32)
    m1 = jnp.max(logits, axis=-1, keepdims=True)
    eq1 = logits == m1
    c1 = jnp.dot(eq1.astype(f32), U8, preferred_element_type=f32)
    f1 = jnp.logical_and(eq1, c1 == 0.0)
    i1 = jnp.dot(f1.astype(f32), icol, preferred_element_type=f32)
    lm = jnp.where(f1, -jnp.inf, logits)
    m2 = jnp.max(lm, axis=-1, keepdims=True)
    eq2 = lm == m2
    c2 = jnp.dot(eq2.astype(f32), U8, preferred_element_type=f32)
    f2 = jnp.logical_and(eq2, c2 == 0.0)
    i2 = jnp.dot(f2.astype(f32), icol, preferred_element_type=f32)
    t = jnp.exp(m2 - m1)
    g0 = 1.0 / (1.0 + t)
    g1 = t * g0
    e0_ref[...] = i1.astype(jnp.int32).reshape(RB // 128, 128)
    e1_ref[...] = i2.astype(jnp.int32).reshape(RB // 128, 128)
    g0_ref[...] = g0.reshape(RB // H, H)
    g1_ref[...] = g1.reshape(RB // H, H)


def _router(y16, Wg, half):
    nb = TH // RB
    return pl.pallas_call(
        _router_kernel,
        grid=(nb,),
        in_specs=[
            pl.BlockSpec((RB, HD), lambda i, h=half, n=nb: (i + h * n, 0)),
            pl.BlockSpec((HD, E), lambda i: (0, 0)),
        ],
        out_specs=[
            pl.BlockSpec((RB // 128, 128), lambda i: (i, 0)),
            pl.BlockSpec((RB // 128, 128), lambda i: (i, 0)),
            pl.BlockSpec((RB // H, H), lambda i: (i, 0)),
            pl.BlockSpec((RB // H, H), lambda i: (i, 0)),
        ],
        out_shape=[
            jax.ShapeDtypeStruct((TH // 128, 128), jnp.int32),
            jax.ShapeDtypeStruct((TH // 128, 128), jnp.int32),
            jax.ShapeDtypeStruct((SH, E), jnp.float32),
            jax.ShapeDtypeStruct((SH, E), jnp.float32),
        ],
    )(y16, Wg)


_NR = TH // 128


def _make_sortmeta_kernel():
    def kern(e0_ref, e1_ref, d0_ref, d1_ref, be_ref):
        f32 = jnp.float32
        ir = jax.lax.broadcasted_iota(jnp.int32, (128, 128), 0)
        ic = jax.lax.broadcasted_iota(jnp.int32, (128, 128), 1)
        U = (ir < ic).astype(f32)
        lr = jax.lax.broadcasted_iota(jnp.int32, (16, 16), 0)
        lc = jax.lax.broadcasted_iota(jnp.int32, (16, 16), 1)
        L16 = (lr > lc).astype(f32)

        e0 = e0_ref[...]
        e1 = e1_ref[...]

        counts = []
        for e in range(E):
            m = (e0 == e).astype(f32) + (e1 == e).astype(f32)
            counts.append(jnp.sum(m))
        starts = []
        s = jnp.float32(0.0)
        for e in range(E):
            starts.append(s)
            s = s + jnp.ceil(counts[e] / BM) * BM

        ib = jax.lax.broadcasted_iota(jnp.int32, (1, 256), 1).astype(f32) * BM
        be = jnp.zeros((1, 256), jnp.int32)
        for e in range(E):
            be = be + (ib >= starts[e]).astype(jnp.int32)
        be_ref[...] = jnp.where(ib < s, be - 1, -1)

        carry = [jnp.float32(0.0)] * E
        for c in range(_NR // 16):
            sl = slice(16 * c, 16 * c + 16)
            e0c = e0[sl, :]
            e1c = e1[sl, :]
            d0c = jnp.zeros((16, 128), f32)
            d1c = jnp.zeros((16, 128), f32)
            for e in range(E):
                m = (e0c == e).astype(f32) + (e1c == e).astype(f32)
                lane_excl = jnp.dot(m, U, preferred_element_type=f32)
                rowtot = jnp.sum(m, axis=1, keepdims=True)
                row_excl = jnp.dot(L16, rowtot, preferred_element_type=f32)
                slot = starts[e] + carry[e] + row_excl + lane_excl
                d0c = d0c + jnp.where(e0c == e, slot, 0.0)
                d1c = d1c + jnp.where(e1c == e, slot, 0.0)
                carry[e] = carry[e] + jnp.sum(m)
            d0_ref[sl, :] = d0c.astype(jnp.int32)
            d1_ref[sl, :] = d1c.astype(jnp.int32)
    return kern


def _sortmeta(e0, e1):
    return pl.pallas_call(
        _make_sortmeta_kernel(),
        grid=(1,),
        in_specs=[
            pl.BlockSpec((_NR, 128), lambda i: (0, 0)),
            pl.BlockSpec((_NR, 128), lambda i: (0, 0)),
        ],
        out_specs=[
            pl.BlockSpec((_NR, 128), lambda i: (0, 0)),
            pl.BlockSpec((_NR, 128), lambda i: (0, 0)),
            pl.BlockSpec((1, 256), lambda i: (0, 0)),
        ],
        out_shape=[
            jax.ShapeDtypeStruct((_NR, 128), jnp.int32),
            jax.ShapeDtypeStruct((_NR, 128), jnp.int32),
            jax.ShapeDtypeStruct((1, 256), jnp.int32),
        ],
    )(e0, e1)


def _make_dispatch_body(half):
    def body(y_hbm, d0_hbm, d1_hbm, yg_hbm, ybuf, d0b, d1b, sem):
        wid = lax.axis_index("s") * 2 + lax.axis_index("c")
        base = half * TH + wid * TPW
        pltpu.sync_copy(d0_hbm.at[pl.ds(wid * NCH, NCH)], d0b)
        pltpu.sync_copy(d1_hbm.at[pl.ds(wid * NCH, NCH)], d1b)
        pltpu.sync_copy(y_hbm.at[pl.ds(base, TPW)], ybuf)
        cps = []
        for j in range(NCH):
            rows = ybuf.at[pl.ds(j * CH, CH)]
            cps.append(pltpu.async_copy(rows, yg_hbm.at[d0b.at[j]], sem))
            cps.append(pltpu.async_copy(rows, yg_hbm.at[d1b.at[j]], sem))
        for cp in cps:
            cp.wait()
    return body


def _dispatch(y16, d0, d1, half):
    mesh = plsc.VectorSubcoreMesh(core_axis_name="c", subcore_axis_name="s")
    kfn = functools.partial(
        pl.kernel,
        out_type=jax.ShapeDtypeStruct((PS, HD), jnp.float32),
        mesh=mesh,
        scratch_types=[
            pltpu.VMEM((TPW, HD), jnp.float32),
            pltpu.VMEM((NCH, CH), jnp.int32),
            pltpu.VMEM((NCH, CH), jnp.int32),
            pltpu.SemaphoreType.DMA,
        ],
    )(_make_dispatch_body(half))
    return kfn(y16, d0, d1)


def _ffn_kernel(be_ref, yg_ref, w1_ref, b1_ref, w2_ref, b2_ref, o_ref):
    i = pl.program_id(0)

    @pl.when(be_ref[0, i] >= 0)
    def _():
        bf16 = jnp.bfloat16
        ygb = yg_ref[...].astype(bf16)
        h = _gelu(
            jnp.dot(ygb, w1_ref[0].astype(bf16),
                    preferred_element_type=jnp.float32)
            + b1_ref[0]
        )
        o_ref[...] = (
            jnp.dot(h.astype(bf16), w2_ref[0].astype(bf16),
                    preferred_element_type=jnp.float32)
            + b2_ref[0]
        )


def _grouped_ffn(be, yg, W1, b1, W2, b2):
    def we(i, be):
        return jnp.maximum(be[0, i], 0)

    grid_spec = pltpu.PrefetchScalarGridSpec(
        num_scalar_prefetch=1,
        grid=(NBH,),
        in_specs=[
            pl.BlockSpec((BM, HD), lambda i, be: (i, 0)),
            pl.BlockSpec((1, HD, F), lambda i, be: (we(i, be), 0, 0)),
            pl.BlockSpec((1, 1, F), lambda i, be: (we(i, be), 0, 0)),
            pl.BlockSpec((1, F, HD), lambda i, be: (we(i, be), 0, 0)),
            pl.BlockSpec((1, 1, HD), lambda i, be: (we(i, be), 0, 0)),
        ],
        out_specs=pl.BlockSpec((BM, HD), lambda i, be: (i, 0)),
    )
    return pl.pallas_call(
        _ffn_kernel,
        grid_spec=grid_spec,
        out_shape=jax.ShapeDtypeStruct((PS, HD), jnp.float32),
    )(be, yg, W1, b1.reshape(E, 1, F), W2, b2.reshape(E, 1, HD))


def _combine_body(eo_hbm, d0_hbm, d1_hbm, r0_hbm, r1_hbm,
                  d0b, d1b, r0buf, r1buf, sem, wsem):
    wid = lax.axis_index("s") * 2 + lax.axis_index("c")
    base = wid * TPW
    pltpu.sync_copy(d0_hbm.at[pl.ds(wid * NCH, NCH)], d0b)
    pltpu.sync_copy(d1_hbm.at[pl.ds(wid * NCH, NCH)], d1b)
    cps = []
    for j in range(NCH):
        dst = pl.ds(j * CH, CH)
        cps.append(pltpu.async_copy(eo_hbm.at[d0b.at[j]], r0buf.at[dst], sem))
        cps.append(pltpu.async_copy(eo_hbm.at[d1b.at[j]], r1buf.at[dst], sem))
    for cp in cps:
        cp.wait()
    out_sl = pl.ds(base, TPW)
    w0 = pltpu.async_copy(r0buf, r0_hbm.at[out_sl], wsem)
    w1 = pltpu.async_copy(r1buf, r1_hbm.at[out_sl], wsem)
    w0.wait()
    w1.wait()


def _combine(eo, d0, d1):
    mesh = plsc.VectorSubcoreMesh(core_axis_name="c", subcore_axis_name="s")
    kfn = functools.partial(
        pl.kernel,
        out_type=[
            jax.ShapeDtypeStruct((TH, HD), jnp.float32),
            jax.ShapeDtypeStruct((TH, HD), jnp.float32),
        ],
        mesh=mesh,
        scratch_types=[
            pltpu.VMEM((NCH, CH), jnp.int32),
            pltpu.VMEM((NCH, CH), jnp.int32),
            pltpu.VMEM((TPW, HD), jnp.float32),
            pltpu.VMEM((TPW, HD), jnp.float32),
            pltpu.SemaphoreType.DMA,
            pltpu.SemaphoreType.DMA,
        ],
    )(_combine_body)
    return kfn(eo, d0, d1)


def _merge_kernel(r0_ref, r1_ref, g0_ref, g1_ref, w_ref, b_ref, o_ref):
    bm = o_ref.shape[0]
    ih = jax.lax.broadcasted_iota(jnp.int32, (E, D), 0)
    ij = jax.lax.broadcasted_iota(jnp.int32, (E, D), 1)
    expand = (ij // HD == ih).astype(jnp.float32)
    g0w = jnp.dot(g0_ref[...], expand, preferred_element_type=jnp.float32)
    g1w = jnp.dot(g1_ref[...], expand, preferred_element_type=jnp.float32)
    r0 = r0_ref[...].reshape(bm, D)
    r1 = r1_ref[...].reshape(bm, D)
    ym = g0w * r0 + g1w * r1
    o_ref[...] = (
        jnp.dot(ym.astype(jnp.bfloat16), w_ref[...].astype(jnp.bfloat16),
                preferred_element_type=jnp.float32)
        + b_ref[...]
    )


def _merge(prev, r0, r1, g0, g1, w, b, half, bm=512):
    in_specs = [
        pl.BlockSpec((bm * H, HD), lambda i: (i, 0)),
        pl.BlockSpec((bm * H, HD), lambda i: (i, 0)),
        pl.BlockSpec((bm, E), lambda i: (i, 0)),
        pl.BlockSpec((bm, E), lambda i: (i, 0)),
        pl.BlockSpec((D, D), lambda i: (0, 0)),
        pl.BlockSpec((1, D), lambda i: (0, 0)),
    ]
    args = (r0, r1, g0, g1, w, b.reshape(1, D))
    kern = _merge_kernel
    aliases = {}
    if prev is not None:
        in_specs = [pl.BlockSpec(memory_space=pl.ANY)] + in_specs
        args = (prev,) + args
        kern = lambda p, *rest: _merge_kernel(*rest)
        aliases = {0: 0}
    return pl.pallas_call(
        kern,
        grid=(SH // bm,),
        in_specs=in_specs,
        out_specs=pl.BlockSpec(
            (bm, D), lambda i, h=half, n=SH // bm: (i + h * n, 0)
        ),
        out_shape=jax.ShapeDtypeStruct((S, D), jnp.float32),
        input_output_aliases=aliases,
    )(*args)


def kernel(x, W_mh, b_mh, Wg, W1, b1, W2, b2, W_merge, b_merge):
    xm = x.reshape(S, D)
    y16 = _mh_proj(xm, W_mh, b_mh)

    parts = []
    for h in range(2):
        e0, e1, g0, g1 = _router(y16, Wg, h)
        d0, d1, be = _sortmeta(e0, e1)
        yg = _dispatch(y16, d0, d1, h)
        eo = _grouped_ffn(be, yg, W1, b1, W2, b2)
        r0, r1 = _combine(eo, d0, d1)
        parts.append((r0, r1, g0, g1))

    out = None
    for h in range(2):
        r0, r1, g0, g1 = parts[h]
        out = _merge(out, r0, r1, g0, g1, W_merge, b_merge, h)
    return out.reshape(B, S, D)


# speedup vs baseline: 1.0040x; 1.0040x over previous
import functools

import jax
import jax.numpy as jnp
from jax import lax
from jax.experimental import pallas as pl
from jax.experimental.pallas import tpu as pltpu
from jax.experimental.pallas import tpu_sc as plsc

B = 1
S = 2048
D = 1024
H = 8
HD = D // H
T = S * H
E = 8
K = 2
F = 512

TH = T // 2
SH = S // 2
BM = 1024
NBH = TH * K // BM + E
PS = NBH * BM
NW = 32
TPW = TH // NW
CH = 128
NCH = TPW // CH
RB = 4096

_GELU_C = 0.7978845608028654
_GC1 = -2.0 * _GELU_C * 1.4426950408889634
_GC2 = _GC1 * 0.044715


def _gelu(x):
    t = jnp.exp2((_GC1 + _GC2 * (x * x)) * x)
    return x / (1.0 + t)


def _mh_kernel(x_ref, w_ref, b_ref, o_ref):
    y = (
        jnp.dot(x_ref[...], w_ref[...], preferred_element_type=jnp.float32)
        + b_ref[...]
    )
    o_ref[...] = y.reshape(o_ref.shape)


def _mh_proj(x, w, b, bm=512):
    return pl.pallas_call(
        _mh_kernel,
        grid=(S // bm,),
        in_specs=[
            pl.BlockSpec((bm, D), lambda i: (i, 0)),
            pl.BlockSpec((D, D), lambda i: (0, 0)),
            pl.BlockSpec((1, D), lambda i: (0, 0)),
        ],
        out_specs=pl.BlockSpec((bm * H, HD), lambda i: (i, 0)),
        out_shape=jax.ShapeDtypeStruct((T, HD), jnp.float32),
    )(x, w, b.reshape(1, D))


def _router_kernel(y_ref, wg_ref, e0_ref, e1_ref, g0_ref, g1_ref):
    y = y_ref[...]
    f32 = jnp.float32
    logits = jnp.dot(y, wg_ref[...], preferred_element_type=f32)
    U8 = (jax.lax.broadcasted_iota(jnp.int32, (E, E), 0)
          < jax.lax.broadcasted_iota(jnp.int32, (E, E), 1)).astype(f32)
    icol = jax.lax.broadcasted_iota(jnp.int32, (E, 1), 0).astype(f32)
    m1 = jnp.max(logits, axis=-1, keepdims=True)
    eq1 = logits == m1
    c1 = jnp.dot(eq1.astype(f32), U8, preferred_element_type=f32)
    f1 = jnp.logical_and(eq1, c1 == 0.0)
    i1 = jnp.dot(f1.astype(f32), icol, preferred_element_type=f32)
    lm = jnp.where(f1, -jnp.inf, logits)
    m2 = jnp.max(lm, axis=-1, keepdims=True)
    eq2 = lm == m2
    c2 = jnp.dot(eq2.astype(f32), U8, preferred_element_type=f32)
    f2 = jnp.logical_and(eq2, c2 == 0.0)
    i2 = jnp.dot(f2.astype(f32), icol, preferred_element_type=f32)
    t = jnp.exp(m2 - m1)
    g0 = 1.0 / (1.0 + t)
    g1 = t * g0
    e0_ref[...] = i1.astype(jnp.int32).reshape(RB // 128, 128)
    e1_ref[...] = i2.astype(jnp.int32).reshape(RB // 128, 128)
    g0_ref[...] = g0.reshape(RB // H, H)
    g1_ref[...] = g1.reshape(RB // H, H)


def _router(y16, Wg, half):
    nb = TH // RB
    return pl.pallas_call(
        _router_kernel,
        grid=(nb,),
        in_specs=[
            pl.BlockSpec((RB, HD), lambda i, h=half, n=nb: (i + h * n, 0)),
            pl.BlockSpec((HD, E), lambda i: (0, 0)),
        ],
        out_specs=[
            pl.BlockSpec((RB // 128, 128), lambda i: (i, 0)),
            pl.BlockSpec((RB // 128, 128), lambda i: (i, 0)),
            pl.BlockSpec((RB // H, H), lambda i: (i, 0)),
            pl.BlockSpec((RB // H, H), lambda i: (i, 0)),
        ],
        out_shape=[
            jax.ShapeDtypeStruct((TH // 128, 128), jnp.int32),
            jax.ShapeDtypeStruct((TH // 128, 128), jnp.int32),
            jax.ShapeDtypeStruct((SH, E), jnp.float32),
            jax.ShapeDtypeStruct((SH, E), jnp.float32),
        ],
    )(y16, Wg)


_NR = TH // 128


def _make_sortmeta_kernel():
    def kern(e0_ref, e1_ref, d0_ref, d1_ref, be_ref):
        f32 = jnp.float32
        ir = jax.lax.broadcasted_iota(jnp.int32, (128, 128), 0)
        ic = jax.lax.broadcasted_iota(jnp.int32, (128, 128), 1)
        U = (ir < ic).astype(f32)
        lr = jax.lax.broadcasted_iota(jnp.int32, (16, 16), 0)
        lc = jax.lax.broadcasted_iota(jnp.int32, (16, 16), 1)
        L16 = (lr > lc).astype(f32)

        e0 = e0_ref[...]
        e1 = e1_ref[...]

        counts = []
        for e in range(E):
            m = (e0 == e).astype(f32) + (e1 == e).astype(f32)
            counts.append(jnp.sum(m))
        starts = []
        s = jnp.float32(0.0)
        for e in range(E):
            starts.append(s)
            s = s + jnp.ceil(counts[e] / BM) * BM

        ib = jax.lax.broadcasted_iota(jnp.int32, (1, 256), 1).astype(f32) * BM
        be = jnp.zeros((1, 256), jnp.int32)
        for e in range(E):
            be = be + (ib >= starts[e]).astype(jnp.int32)
        be_ref[...] = jnp.where(ib < s, be - 1, -1)

        carry = [jnp.float32(0.0)] * E
        for c in range(_NR // 16):
            sl = slice(16 * c, 16 * c + 16)
            e0c = e0[sl, :]
            e1c = e1[sl, :]
            d0c = jnp.zeros((16, 128), f32)
            d1c = jnp.zeros((16, 128), f32)
            for e in range(E):
                m = (e0c == e).astype(f32) + (e1c == e).astype(f32)
                lane_excl = jnp.dot(m, U, preferred_element_type=f32)
                rowtot = jnp.sum(m, axis=1, keepdims=True)
                row_excl = jnp.dot(L16, rowtot, preferred_element_type=f32)
                slot = starts[e] + carry[e] + row_excl + lane_excl
                d0c = d0c + jnp.where(e0c == e, slot, 0.0)
                d1c = d1c + jnp.where(e1c == e, slot, 0.0)
                carry[e] = carry[e] + jnp.sum(m)
            d0_ref[sl, :] = d0c.astype(jnp.int32)
            d1_ref[sl, :] = d1c.astype(jnp.int32)
    return kern


def _sortmeta(e0, e1):
    return pl.pallas_call(
        _make_sortmeta_kernel(),
        grid=(1,),
        in_specs=[
            pl.BlockSpec((_NR, 128), lambda i: (0, 0)),
            pl.BlockSpec((_NR, 128), lambda i: (0, 0)),
        ],
        out_specs=[
            pl.BlockSpec((_NR, 128), lambda i: (0, 0)),
            pl.BlockSpec((_NR, 128), lambda i: (0, 0)),
            pl.BlockSpec((1, 256), lambda i: (0, 0)),
        ],
        out_shape=[
            jax.ShapeDtypeStruct((_NR, 128), jnp.int32),
            jax.ShapeDtypeStruct((_NR, 128), jnp.int32),
            jax.ShapeDtypeStruct((1, 256), jnp.int32),
        ],
    )(e0, e1)


def _make_dispatch_body(half):
    def body(y_hbm, d0_hbm, d1_hbm, yg_hbm, ybuf, d0b, d1b, sem):
        wid = lax.axis_index("s") * 2 + lax.axis_index("c")
        base = half * TH + wid * TPW
        pltpu.sync_copy(d0_hbm.at[pl.ds(wid * NCH, NCH)], d0b)
        pltpu.sync_copy(d1_hbm.at[pl.ds(wid * NCH, NCH)], d1b)
        pltpu.sync_copy(y_hbm.at[pl.ds(base, TPW)], ybuf)
        cps = []
        for j in range(NCH):
            rows = ybuf.at[pl.ds(j * CH, CH)]
            cps.append(pltpu.async_copy(rows, yg_hbm.at[d0b.at[j]], sem))
            cps.append(pltpu.async_copy(rows, yg_hbm.at[d1b.at[j]], sem))
        for cp in cps:
            cp.wait()
    return body


def _dispatch(y16, d0, d1, half):
    mesh = plsc.VectorSubcoreMesh(core_axis_name="c", subcore_axis_name="s")
    kfn = functools.partial(
        pl.kernel,
        out_type=jax.ShapeDtypeStruct((PS, HD), jnp.float32),
        mesh=mesh,
        scratch_types=[
            pltpu.VMEM((TPW, HD), jnp.float32),
            pltpu.VMEM((NCH, CH), jnp.int32),
            pltpu.VMEM((NCH, CH), jnp.int32),
            pltpu.SemaphoreType.DMA,
        ],
    )(_make_dispatch_body(half))
    return kfn(y16, d0, d1)


def _ffn_kernel(be_ref, yg_ref, w1_ref, b1_ref, w2_ref, b2_ref, o_ref):
    i = pl.program_id(0)

    @pl.when(be_ref[0, i] >= 0)
    def _():
        bf16 = jnp.bfloat16
        ygb = yg_ref[...].astype(bf16)
        h = _gelu(
            jnp.dot(ygb, w1_ref[0].astype(bf16),
                    preferred_element_type=jnp.float32)
            + b1_ref[0]
        )
        o_ref[...] = (
            jnp.dot(h.astype(bf16), w2_ref[0].astype(bf16),
                    preferred_element_type=jnp.float32)
            + b2_ref[0]
        )


def _grouped_ffn(be, yg, W1, b1, W2, b2):
    def we(i, be):
        return jnp.maximum(be[0, i], 0)

    grid_spec = pltpu.PrefetchScalarGridSpec(
        num_scalar_prefetch=1,
        grid=(NBH,),
        in_specs=[
            pl.BlockSpec((BM, HD), lambda i, be: (i, 0)),
            pl.BlockSpec((1, HD, F), lambda i, be: (we(i, be), 0, 0)),
            pl.BlockSpec((1, 1, F), lambda i, be: (we(i, be), 0, 0)),
            pl.BlockSpec((1, F, HD), lambda i, be: (we(i, be), 0, 0)),
            pl.BlockSpec((1, 1, HD), lambda i, be: (we(i, be), 0, 0)),
        ],
        out_specs=pl.BlockSpec((BM, HD), lambda i, be: (i, 0)),
    )
    return pl.pallas_call(
        _ffn_kernel,
        grid_spec=grid_spec,
        out_shape=jax.ShapeDtypeStruct((PS, HD), jnp.float32),
    )(be, yg, W1, b1.reshape(E, 1, F), W2, b2.reshape(E, 1, HD))


def _combine_body(eo_hbm, d0_hbm, d1_hbm, r0_hbm, r1_hbm,
                  d0b, d1b, r0buf, r1buf, sem, wsem):
    wid = lax.axis_index("s") * 2 + lax.axis_index("c")
    base = wid * TPW
    pltpu.sync_copy(d0_hbm.at[pl.ds(wid * NCH, NCH)], d0b)
    pltpu.sync_copy(d1_hbm.at[pl.ds(wid * NCH, NCH)], d1b)
    cps = []
    for j in range(NCH):
        dst = pl.ds(j * CH, CH)
        cps.append(pltpu.async_copy(eo_hbm.at[d0b.at[j]], r0buf.at[dst], sem))
        cps.append(pltpu.async_copy(eo_hbm.at[d1b.at[j]], r1buf.at[dst], sem))
    for cp in cps:
        cp.wait()
    out_sl = pl.ds(base, TPW)
    w0 = pltpu.async_copy(r0buf, r0_hbm.at[out_sl], wsem)
    w1 = pltpu.async_copy(r1buf, r1_hbm.at[out_sl], wsem)
    w0.wait()
    w1.wait()


def _combine(eo, d0, d1):
    mesh = plsc.VectorSubcoreMesh(core_axis_name="c", subcore_axis_name="s")
    kfn = functools.partial(
        pl.kernel,
        out_type=[
            jax.ShapeDtypeStruct((TH, HD), jnp.float32),
            jax.ShapeDtypeStruct((TH, HD), jnp.float32),
        ],
        mesh=mesh,
        scratch_types=[
            pltpu.VMEM((NCH, CH), jnp.int32),
            pltpu.VMEM((NCH, CH), jnp.int32),
            pltpu.VMEM((TPW, HD), jnp.float32),
            pltpu.VMEM((TPW, HD), jnp.float32),
            pltpu.SemaphoreType.DMA,
            pltpu.SemaphoreType.DMA,
        ],
    )(_combine_body)
    return kfn(eo, d0, d1)


def _merge_kernel(r0_ref, r1_ref, g0_ref, g1_ref, w_ref, b_ref, o_ref):
    bm = o_ref.shape[0]
    ih = jax.lax.broadcasted_iota(jnp.int32, (E, D), 0)
    ij = jax.lax.broadcasted_iota(jnp.int32, (E, D), 1)
    expand = (ij // HD == ih).astype(jnp.float32)
    g0w = jnp.dot(g0_ref[...], expand, preferred_element_type=jnp.float32)
    g1w = jnp.dot(g1_ref[...], expand, preferred_element_type=jnp.float32)
    r0 = r0_ref[...].reshape(bm, D)
    r1 = r1_ref[...].reshape(bm, D)
    ym = g0w * r0 + g1w * r1
    o_ref[...] = (
        jnp.dot(ym.astype(jnp.bfloat16), w_ref[...].astype(jnp.bfloat16),
                preferred_element_type=jnp.float32)
        + b_ref[...]
    )


def _merge(prev, r0, r1, g0, g1, w, b, half, bm=512):
    in_specs = [
        pl.BlockSpec((bm * H, HD), lambda i: (i, 0)),
        pl.BlockSpec((bm * H, HD), lambda i: (i, 0)),
        pl.BlockSpec((bm, E), lambda i: (i, 0)),
        pl.BlockSpec((bm, E), lambda i: (i, 0)),
        pl.BlockSpec((D, D), lambda i: (0, 0)),
        pl.BlockSpec((1, D), lambda i: (0, 0)),
    ]
    args = (r0, r1, g0, g1, w, b.reshape(1, D))
    kern = _merge_kernel
    aliases = {}
    if prev is not None:
        in_specs = [pl.BlockSpec(memory_space=pl.ANY)] + in_specs
        args = (prev,) + args
        kern = lambda p, *rest: _merge_kernel(*rest)
        aliases = {0: 0}
    return pl.pallas_call(
        kern,
        grid=(SH // bm,),
        in_specs=in_specs,
        out_specs=pl.BlockSpec(
            (bm, D), lambda i, h=half, n=SH // bm: (i + h * n, 0)
        ),
        out_shape=jax.ShapeDtypeStruct((S, D), jnp.float32),
        input_output_aliases=aliases,
    )(*args)


def kernel(x, W_mh, b_mh, Wg, W1, b1, W2, b2, W_merge, b_merge):
    xm = x.reshape(S, D)
    y16 = _mh_proj(xm, W_mh, b_mh)

    parts = []
    for h in range(2):
        e0, e1, g0, g1 = _router(y16, Wg, h)
        d0, d1, be = _sortmeta(e0, e1)
        yg = _dispatch(y16, d0, d1, h)
        eo = _grouped_ffn(be, yg, W1, b1, W2, b2)
        r0, r1 = _combine(eo, d0, d1)
        parts.append((r0, r1, g0, g1))

    out = None
    for h in range(2):
        r0, r1, g0, g1 = parts[h]
        out = _merge(out, r0, r1, g0, g1, W_merge, b_merge, h)
    return out.reshape(B, S, D)
